# baseline (speedup 1.0000x reference)
import functools

import jax
import jax.numpy as jnp
from jax import lax
from jax.experimental import pallas as pl
from jax.experimental.pallas import tpu as pltpu
from jax.experimental.pallas import tpu_sc as plsc

NC = 2
NS = 16
NW = NC * NS

CB = 4
NBUF = 2


def _gather_call(batch, seq, dim):
    bpw = batch // NW
    nch = bpw // CB
    s0 = (seq // 2 + 7) // 8 * 8

    mesh = plsc.VectorSubcoreMesh(core_axis_name="c", subcore_axis_name="s")

    @functools.partial(
        pl.kernel,
        out_type=jax.ShapeDtypeStruct((batch, seq, dim), jnp.float32),
        mesh=mesh,
        compiler_params=pltpu.CompilerParams(use_tc_tiling_on_sc=False),
        scratch_types=[
            pltpu.VMEM((bpw, seq), jnp.int32),
            pltpu.VMEM((NBUF, CB, seq, dim), jnp.float32),
            pltpu.SemaphoreType.DMA,
            pltpu.SemaphoreType.DMA,
            pltpu.SemaphoreType.DMA,
            pltpu.SemaphoreType.DMA,
        ],
    )
    def gather_kernel(ids_hbm, table_hbm, out_hbm, idx_v, rows_v,
                      gsem0, gsem1, osem0, osem1):
        gsems = (gsem0, gsem1)
        osems = (osem0, osem1)
        wid = lax.axis_index("s") * NC + lax.axis_index("c")
        b0 = wid * bpw
        pltpu.sync_copy(ids_hbm.at[pl.ds(b0, bpw)], idx_v)

        def issue_gathers(c, b, start):
            for br in range(CB):
                r = c * CB + br
                for (o, ln) in ((0, s0), (s0, seq - s0)):
                    src = table_hbm.at[idx_v.at[r, pl.ds(o, ln)]]
                    dst = rows_v.at[b, br].at[pl.ds(o, ln)]
                    if start:
                        pltpu.async_copy(src, dst, gsems[b])
                    else:
                        pltpu.make_async_copy(src, dst, gsems[b]).wait()

        def finish_chunk(c, b):
            issue_gathers(c, b, start=False)
            pltpu.async_copy(rows_v.at[b],
                             out_hbm.at[pl.ds(b0 + c * CB, CB)], osems[b])

        def chunk_body(c, _):
            for b in range(NBUF):
                @pl.when(lax.rem(c, NBUF) == b)
                def _(b=b):
                    @pl.when(c >= NBUF)
                    def _():
                        pltpu.make_async_copy(
                            rows_v.at[b],
                            out_hbm.at[pl.ds(b0 + (c - NBUF) * CB, CB)],
                            osems[b],
                        ).wait()
                    issue_gathers(c, b, start=True)
                    @pl.when(c >= 1)
                    def _():
                        finish_chunk(c - 1, 1 - b)
            return 0

        lax.fori_loop(0, nch, chunk_body, 0)
        finish_chunk(nch - 1, (nch - 1) % NBUF)
        for b in range(NBUF):
            c = nch - NBUF + b
            pltpu.make_async_copy(
                rows_v.at[c % NBUF],
                out_hbm.at[pl.ds(b0 + c * CB, CB)],
                osems[c % NBUF],
            ).wait()

    return gather_kernel


def kernel(input_ids, base_table, special_A, special_B, lookup_A, lookup_B):
    del lookup_A, lookup_B
    batch, seq = input_ids.shape
    dim = base_table.shape[1]
    table = jnp.concatenate([base_table, special_A, special_B], axis=0)
    return _gather_call(batch, seq, dim)(input_ids, table)
